# baseline (speedup 1.0000x reference)
import jax
import jax.numpy as jnp
from jax.experimental import pallas as pl
from jax.experimental.pallas import tpu as pltpu

DIM = 768
ROWS_PER_BLOCK = 256


def _body(q_ref, m_ref, at_ref, b_ref, o_ref):
    q = q_ref[...]
    qb = q.astype(jnp.bfloat16)
    p = 0.4 * q + jnp.dot(qb, m_ref[...], preferred_element_type=jnp.float32)
    t = jnp.dot(qb, at_ref[...], preferred_element_type=jnp.float32)
    rinv = jax.lax.rsqrt(
        jnp.maximum(jnp.sum(q * q, axis=1, keepdims=True), 1e-24))
    c = (t * rinv).astype(jnp.bfloat16)
    y = p + jnp.dot(c, b_ref[...], preferred_element_type=jnp.float32)
    yn = jax.lax.rsqrt(
        jnp.maximum(jnp.sum(y * y, axis=1, keepdims=True), 1e-24))
    o_ref[...] = y * yn


def kernel(Q, axes, strength, metric):
    B, S, D = Q.shape
    n = B * S
    Q2 = Q.reshape(n, D)
    an = axes / jnp.maximum(
        jnp.linalg.norm(axes, axis=-1, keepdims=True), 1e-12)
    s = jax.nn.relu(strength) + 1e-06
    w = s / jnp.sum(s)
    m_pert = (0.4 * (metric - jnp.eye(D, dtype=metric.dtype))).astype(
        jnp.bfloat16)
    at = an.T.astype(jnp.bfloat16)
    bmat = ((0.6 * w)[:, None] * an).astype(jnp.bfloat16)

    grid = (n // ROWS_PER_BLOCK,)
    out = pl.pallas_call(
        _body,
        grid=grid,
        in_specs=[
            pl.BlockSpec((ROWS_PER_BLOCK, D), lambda i: (i, 0)),
            pl.BlockSpec((D, D), lambda i: (0, 0)),
            pl.BlockSpec((D, axes.shape[0]), lambda i: (0, 0)),
            pl.BlockSpec((axes.shape[0], D), lambda i: (0, 0)),
        ],
        out_specs=pl.BlockSpec((ROWS_PER_BLOCK, D), lambda i: (i, 0)),
        out_shape=jax.ShapeDtypeStruct((n, D), jnp.float32),
        compiler_params=pltpu.CompilerParams(
            dimension_semantics=("arbitrary",),
        ),
    )(Q2, m_pert, at, bmat)
    return out.reshape(B, S, D)


# speedup vs baseline: 1.3826x; 1.3826x over previous
import jax
import jax.numpy as jnp
from jax.experimental import pallas as pl
from jax.experimental.pallas import tpu as pltpu

DIM = 768
ROWS_PER_BLOCK = 512


def _body(q_ref, m_ref, at_ref, b_ref, o_ref):
    q = q_ref[...]
    qb = q.astype(jnp.bfloat16)
    p = 0.4 * q + jnp.dot(qb, m_ref[...], preferred_element_type=jnp.float32)
    t = jnp.dot(qb, at_ref[...], preferred_element_type=jnp.float32)
    rinv = jax.lax.rsqrt(
        jnp.maximum(jnp.sum(q * q, axis=1, keepdims=True), 1e-24))
    c = (t * rinv).astype(jnp.bfloat16)
    y = p + jnp.dot(c, b_ref[...], preferred_element_type=jnp.float32)
    yn = jax.lax.rsqrt(
        jnp.maximum(jnp.sum(y * y, axis=1, keepdims=True), 1e-24))
    o_ref[...] = y * yn


def kernel(Q, axes, strength, metric):
    B, S, D = Q.shape
    n = B * S
    Q2 = Q.reshape(n, D)
    an = axes / jnp.maximum(
        jnp.linalg.norm(axes, axis=-1, keepdims=True), 1e-12)
    s = jax.nn.relu(strength) + 1e-06
    w = s / jnp.sum(s)
    m_pert = (0.4 * (metric - jnp.eye(D, dtype=metric.dtype))).astype(
        jnp.bfloat16)
    at = an.T.astype(jnp.bfloat16)
    bmat = ((0.6 * w)[:, None] * an).astype(jnp.bfloat16)

    grid = (n // ROWS_PER_BLOCK,)
    out = pl.pallas_call(
        _body,
        grid=grid,
        in_specs=[
            pl.BlockSpec((ROWS_PER_BLOCK, D), lambda i: (i, 0)),
            pl.BlockSpec((D, D), lambda i: (0, 0)),
            pl.BlockSpec((D, axes.shape[0]), lambda i: (0, 0)),
            pl.BlockSpec((axes.shape[0], D), lambda i: (0, 0)),
        ],
        out_specs=pl.BlockSpec((ROWS_PER_BLOCK, D), lambda i: (i, 0)),
        out_shape=jax.ShapeDtypeStruct((n, D), jnp.float32),
        compiler_params=pltpu.CompilerParams(
            dimension_semantics=("arbitrary",),
        ),
    )(Q2, m_pert, at, bmat)
    return out.reshape(B, S, D)
